# baseline (speedup 1.0000x reference)
import functools

import jax
import jax.numpy as jnp
from jax import lax
from jax.experimental import pallas as pl
from jax.experimental.pallas import tpu as pltpu
from jax.experimental.pallas import tpu_sc as plsc

M = 2097152
D = 16
NC, NS = 2, 16
NW = NC * NS
WR = M // NW
CR = 1024
NCHUNK = WR // CR

_mesh = plsc.VectorSubcoreMesh(core_axis_name="c", subcore_axis_name="s")


@functools.partial(
    pl.kernel,
    mesh=_mesh,
    out_type=jax.ShapeDtypeStruct((M, D), jnp.float32),
    compiler_params=pltpu.CompilerParams(use_tc_tiling_on_sc=False),
    scratch_types=[
        pltpu.VMEM((CR, D), jnp.float32),
        pltpu.VMEM((CR, D), jnp.float32),
    ],
)
def _sc_masked_add(a_hbm, b_hbm, o_hbm, a_v, b_v):
    wid = lax.axis_index("s") * NC + lax.axis_index("c")
    base = pl.multiple_of(wid * WR, CR)

    def chunk_body(ci, carry):
        off = pl.multiple_of(base + ci * CR, CR)
        pltpu.sync_copy(a_hbm.at[pl.ds(off, CR)], a_v)
        pltpu.sync_copy(b_hbm.at[pl.ds(off, CR)], b_v)

        def vec_body(i, carry2):
            a = a_v[i]
            b = b_v[i]
            a_v[i] = jnp.where(a == 1.0, 0.0, a) + b
            return carry2

        lax.fori_loop(0, CR, vec_body, 0, unroll=8)
        pltpu.sync_copy(a_v, o_hbm.at[pl.ds(off, CR)])
        return carry

    lax.fori_loop(0, NCHUNK, chunk_body, 0)


def kernel(x_1, x_2):
    return _sc_masked_add(x_1, x_2)


# speedup vs baseline: 24.0111x; 24.0111x over previous
import jax
import jax.numpy as jnp
from jax.experimental import pallas as pl


def _body(a_ref, b_ref, o_ref):
    a = a_ref[...]
    o_ref[...] = jnp.where(a == 1.0, 0.0, a) + b_ref[...]


def kernel(x_1, x_2):
    a = x_1.T
    b = x_2.T
    n = a.shape[1]
    bn = 65536
    out = pl.pallas_call(
        _body,
        grid=(n // bn,),
        in_specs=[
            pl.BlockSpec((16, bn), lambda i: (0, i)),
            pl.BlockSpec((16, bn), lambda i: (0, i)),
        ],
        out_specs=pl.BlockSpec((16, bn), lambda i: (0, i)),
        out_shape=jax.ShapeDtypeStruct((16, n), jnp.float32),
    )(a, b)
    return out.T
